# baseline (speedup 1.0000x reference)
import jax
import jax.numpy as jnp
from jax import lax
from jax.experimental import pallas as pl
from jax.experimental.pallas import tpu as pltpu
from jax.experimental.pallas import tpu_sc as plsc

D_MODEL = 768
MAX_REL = 128
VOCAB = 2 * MAX_REL + 1
S = 512

NC = 2
NS = 16
NW = NC * NS
ROWS_PER_W = S // NW
CHUNK = 64
NCHUNK = S // CHUNK


def _rpe_body(table_hbm, out_hbm, idx_v, rows_v, sem):
    wid = lax.axis_index("s") * NC + lax.axis_index("c")
    i0 = wid * ROWS_PER_W

    def step(t, carry):
        i = i0 + t // NCHUNK
        j0 = (t % NCHUNK) * CHUNK
        for g in range(CHUNK // 16):
            lanes = lax.iota(jnp.int32, 16) + (j0 + g * 16 + MAX_REL)
            idx_v[pl.ds(g * 16, 16)] = jnp.clip(lanes - i, 0, VOCAB - 1)
        pltpu.async_copy(table_hbm.at[idx_v], rows_v, sem).wait()
        pltpu.sync_copy(rows_v, out_hbm.at[pl.ds(i * S + j0, CHUNK)])
        return carry

    lax.fori_loop(0, ROWS_PER_W * NCHUNK, step, 0)


def kernel(seq_len, table):
    out = pl.kernel(
        _rpe_body,
        mesh=plsc.VectorSubcoreMesh(core_axis_name="c", subcore_axis_name="s"),
        out_type=jax.ShapeDtypeStruct((S * S, D_MODEL), jnp.float32),
        scratch_types=[
            pltpu.VMEM((CHUNK,), jnp.int32),
            pltpu.VMEM((CHUNK, D_MODEL), jnp.float32),
            pltpu.SemaphoreType.DMA,
        ],
    )(table)
    return out.reshape(S, S, D_MODEL)
